# baseline (device time: 93243 ns/iter reference)
import jax
import jax.numpy as jnp
from jax import lax
from jax.experimental import pallas as pl
from jax.experimental.pallas import tpu as pltpu

K_FIX = 64
NCH = 8
SCH = 64


def kernel(x, A, B, C):
    Bb, S, D = x.shape
    N = A.shape[1]
    Dh = D // 2

    def body(x_ref, a_ref, b_ref, c_ref, out_ref,
             h_ref, xh_ref, yh_ref, rx_ref,
             seam_send, seam_recv, ch_send, ch_recv,
             stage_sem, end_sem, cr_seam, cr_chunk):
        my_x = lax.axis_index("x")
        my_y = lax.axis_index("y")
        other_x = 1 - my_x
        other_y = 1 - my_y
        d0 = my_y * Dh
        d0_twin = other_y * Dh

        barrier_sem = pltpu.get_barrier_semaphore()
        pl.semaphore_signal(
            barrier_sem, inc=1,
            device_id=(other_x, my_y), device_id_type=pl.DeviceIdType.MESH,
        )
        pl.semaphore_signal(
            barrier_sem, inc=1,
            device_id=(my_x, other_y), device_id_type=pl.DeviceIdType.MESH,
        )
        pl.semaphore_wait(barrier_sem, 2)

        pl.semaphore_signal(
            cr_chunk, inc=1,
            device_id=(my_x, other_y), device_id_type=pl.DeviceIdType.MESH,
        )

        @pl.when(my_x == 1)
        def _():
            pl.semaphore_signal(
                cr_seam, inc=1,
                device_id=(0, my_y), device_id_type=pl.DeviceIdType.MESH,
            )

        pl.semaphore_wait(cr_chunk, 1)

        def stage(b):
            return pltpu.make_async_copy(
                x_ref.at[b, :, pl.ds(d0, Dh)], xh_ref.at[:, b, :],
                stage_sem.at[b],
            )

        for b in range(Bb):
            stage(b).start()

        dAT = jnp.exp(a_ref[:, :]).T
        dAh = jnp.where(
            my_y == 0, dAT[:, :Dh], dAT[:, Dh:]
        ).reshape(1, N, Dh)

        for b in range(Bb):
            stage(b).wait()

        def step(t, h):
            x_t = xh_ref[t]
            b_t = b_ref[:, t, :]
            c_t = c_ref[:, t, :]
            h = h * dAh + x_t[:, None, :] * b_t[:, :, None]
            yh_ref[t] = jnp.sum(h * c_t[:, :, None], axis=1)
            return h

        def chunk_rdma(c):
            return pltpu.make_async_remote_copy(
                src_ref=yh_ref.at[pl.ds(c * SCH, SCH)],
                dst_ref=rx_ref.at[pl.ds(c * SCH, SCH)],
                send_sem=ch_send.at[c], recv_sem=ch_recv.at[c],
                device_id=(my_x, other_y), device_id_type=pl.DeviceIdType.MESH,
            )

        h = jnp.zeros((Bb, N, Dh), jnp.float32)
        for c in range(NCH):
            h = lax.fori_loop(c * SCH, (c + 1) * SCH, step, h)
            if c > 0:
                chunk_rdma(c).start()

        @pl.when(my_x == 0)
        def _():
            h_ref[...] = h
            pl.semaphore_wait(cr_seam, 1)
            seam = pltpu.make_async_remote_copy(
                src_ref=h_ref, dst_ref=h_ref,
                send_sem=seam_send, recv_sem=seam_recv,
                device_id=(1, my_y), device_id_type=pl.DeviceIdType.MESH,
            )
            seam.start()
            seam.wait_send()

        @pl.when(my_x == 1)
        def _():
            seam = pltpu.make_async_remote_copy(
                src_ref=h_ref, dst_ref=h_ref,
                send_sem=seam_send, recv_sem=seam_recv,
                device_id=(0, my_y), device_id_type=pl.DeviceIdType.MESH,
            )
            seam.wait_recv()

            def cstep(j, hc):
                hc = hc * dAh
                c_j = c_ref[:, j, :]
                yh_ref[j] += jnp.sum(hc * c_j[:, :, None], axis=1)
                return hc

            lax.fori_loop(0, K_FIX, cstep, h_ref[...])

        chunk_rdma(0).start()

        def my_copy(b):
            return pltpu.make_async_copy(
                yh_ref.at[:, b, :], out_ref.at[b, :, pl.ds(d0, Dh)],
                end_sem.at[b],
            )

        def twin_copy(b):
            return pltpu.make_async_copy(
                rx_ref.at[:, b, :], out_ref.at[b, :, pl.ds(d0_twin, Dh)],
                end_sem.at[Bb + b],
            )

        for b in range(Bb):
            my_copy(b).start()

        for c in range(NCH):
            chunk_rdma(c).wait_send()
            chunk_rdma(c).wait_recv()
        for b in range(Bb):
            twin_copy(b).start()
        for b in range(Bb):
            my_copy(b).wait()
            twin_copy(b).wait()

    return pl.pallas_call(
        body,
        out_shape=jax.ShapeDtypeStruct((Bb, S, D), jnp.float32),
        in_specs=[pl.BlockSpec(memory_space=pltpu.VMEM)] * 4,
        out_specs=pl.BlockSpec(memory_space=pltpu.VMEM),
        scratch_shapes=[
            pltpu.VMEM((Bb, N, Dh), jnp.float32),
            pltpu.VMEM((S, Bb, Dh), jnp.float32),
            pltpu.VMEM((S, Bb, Dh), jnp.float32),
            pltpu.VMEM((S, Bb, Dh), jnp.float32),
            pltpu.SemaphoreType.DMA,
            pltpu.SemaphoreType.DMA,
            pltpu.SemaphoreType.DMA((NCH,)),
            pltpu.SemaphoreType.DMA((NCH,)),
            pltpu.SemaphoreType.DMA((Bb,)),
            pltpu.SemaphoreType.DMA((2 * Bb,)),
            pltpu.SemaphoreType.REGULAR,
            pltpu.SemaphoreType.REGULAR,
        ],
        compiler_params=pltpu.CompilerParams(collective_id=0),
    )(x, A, B, C)


# device time: 63875 ns/iter; 1.4598x vs baseline; 1.4598x over previous
import jax
import jax.numpy as jnp
from jax import lax
from jax.experimental import pallas as pl
from jax.experimental.pallas import tpu as pltpu

K_FIX = 32
NCH = 16
SCH = 32


def kernel(x, A, B, C):
    Bb, S, D = x.shape
    N = A.shape[1]
    Bh = Bb // 2

    def body(x_ref, a_ref, b_ref, c_ref, out_ref,
             h_ref, seam_send, seam_recv, ch_send, ch_recv,
             cr_seam, cr_chunk):
        my_x = lax.axis_index("x")
        my_y = lax.axis_index("y")
        other_x = 1 - my_x
        other_y = 1 - my_y
        b0 = my_y * Bh

        barrier_sem = pltpu.get_barrier_semaphore()
        pl.semaphore_signal(
            barrier_sem, inc=1,
            device_id=(other_x, my_y), device_id_type=pl.DeviceIdType.MESH,
        )
        pl.semaphore_signal(
            barrier_sem, inc=1,
            device_id=(my_x, other_y), device_id_type=pl.DeviceIdType.MESH,
        )
        pl.semaphore_wait(barrier_sem, 2)

        pl.semaphore_signal(
            cr_chunk, inc=1,
            device_id=(my_x, other_y), device_id_type=pl.DeviceIdType.MESH,
        )

        @pl.when(my_x == 1)
        def _():
            pl.semaphore_signal(
                cr_seam, inc=1,
                device_id=(0, my_y), device_id_type=pl.DeviceIdType.MESH,
            )

        pl.semaphore_wait(cr_chunk, 1)

        dAT = jnp.exp(a_ref[:, :]).T.reshape(1, N, D)

        UNR = 16

        def step(i, h):
            t0 = i * UNR
            xc = x_ref[pl.ds(b0, Bh), pl.ds(t0, UNR), :]
            bc = b_ref[pl.ds(b0, Bh), pl.ds(t0, UNR), :]
            cc = c_ref[pl.ds(b0, Bh), pl.ds(t0, UNR), :]
            ys = []
            for j in range(UNR):
                h = h * dAT + xc[:, j, None, :] * bc[:, j, :, None]
                ys.append(jnp.sum(h * cc[:, j, :, None], axis=1))
            out_ref[pl.ds(b0, Bh), pl.ds(t0, UNR), :] = jnp.stack(ys, axis=1)
            return h

        def chunk_rdma(c):
            return pltpu.make_async_remote_copy(
                src_ref=out_ref.at[pl.ds(b0, Bh), pl.ds(c * SCH, SCH), :],
                dst_ref=out_ref.at[pl.ds(b0, Bh), pl.ds(c * SCH, SCH), :],
                send_sem=ch_send.at[c], recv_sem=ch_recv.at[c],
                device_id=(my_x, other_y), device_id_type=pl.DeviceIdType.MESH,
            )

        h = jnp.zeros((Bh, N, D), jnp.float32)
        for c in range(NCH):
            h = lax.fori_loop(c * SCH // UNR, (c + 1) * SCH // UNR, step, h)
            if c > 0:
                chunk_rdma(c).start()

        @pl.when(my_x == 0)
        def _():
            h_ref[...] = h
            pl.semaphore_wait(cr_seam, 1)
            seam = pltpu.make_async_remote_copy(
                src_ref=h_ref, dst_ref=h_ref,
                send_sem=seam_send, recv_sem=seam_recv,
                device_id=(1, my_y), device_id_type=pl.DeviceIdType.MESH,
            )
            seam.start()
            seam.wait_send()

        @pl.when(my_x == 1)
        def _():
            seam = pltpu.make_async_remote_copy(
                src_ref=h_ref, dst_ref=h_ref,
                send_sem=seam_send, recv_sem=seam_recv,
                device_id=(0, my_y), device_id_type=pl.DeviceIdType.MESH,
            )
            seam.wait_recv()

            def cstep(i, hc):
                j0 = i * UNR
                ccc = c_ref[pl.ds(b0, Bh), pl.ds(j0, UNR), :]
                ys = []
                for j in range(UNR):
                    hc = hc * dAT
                    ys.append(jnp.sum(hc * ccc[:, j, :, None], axis=1))
                out_ref[pl.ds(b0, Bh), pl.ds(j0, UNR), :] += jnp.stack(ys, axis=1)
                return hc

            lax.fori_loop(0, K_FIX // UNR, cstep, h_ref[...])

        chunk_rdma(0).start()

        for c in range(NCH):
            chunk_rdma(c).wait_send()
            chunk_rdma(c).wait_recv()

    return pl.pallas_call(
        body,
        out_shape=jax.ShapeDtypeStruct((Bb, S, D), jnp.float32),
        in_specs=[pl.BlockSpec(memory_space=pltpu.VMEM)] * 4,
        out_specs=pl.BlockSpec(memory_space=pltpu.VMEM),
        scratch_shapes=[
            pltpu.VMEM((Bh, N, D), jnp.float32),
            pltpu.SemaphoreType.DMA,
            pltpu.SemaphoreType.DMA,
            pltpu.SemaphoreType.DMA((NCH,)),
            pltpu.SemaphoreType.DMA((NCH,)),
            pltpu.SemaphoreType.REGULAR,
            pltpu.SemaphoreType.REGULAR,
        ],
        compiler_params=pltpu.CompilerParams(collective_id=0),
    )(x, A, B, C)
